# baseline (speedup 1.0000x reference)
import functools

import jax
import jax.numpy as jnp
from jax import lax
from jax.experimental import pallas as pl
from jax.experimental.pallas import tpu as pltpu
from jax.experimental.pallas import tpu_sc as plsc

N_NODES = 50000
N_PAD = 50048
N_EDGES = 800000
NPASS = 5
CHUNK = 128


def _sc_edge_kernel(asrc0, asrc1, asrc2, asrc3, adst0, adst1, adst2, adst3,
                    esrc, edst, h_out, acc, idx_s, idx_d, idx_st, idx_dt,
                    rows_s, rows_d, rows_st, rows_dt, hbuf, hbuf_t, zbuf):
    c = lax.axis_index("c")
    s = lax.axis_index("s")
    asrc_tabs = (asrc0, asrc1, asrc2, asrc3)
    adst_tabs = (adst0, adst1, adst2, adst3)

    epc = N_EDGES // 2
    ept = epc // 16
    nfull = ept // CHUNK
    tail = ept - nfull * CHUNK
    base = c * epc + s * ept

    rpt = N_PAD // 16
    zrows = 136

    zero16 = jnp.zeros((16,), jnp.float32)
    one16 = jnp.ones((16,), jnp.float32)

    @pl.loop(0, zrows)
    def _(j):
        zbuf[j, pl.ds(0, 16)] = zero16
        zbuf[j, pl.ds(16, 16)] = zero16

    @pl.loop(0, CHUNK)
    def _(j):
        hbuf[j, pl.ds(0, 16)] = one16
        hbuf[j, pl.ds(16, 16)] = one16

    @pl.loop(0, tail)
    def _(j):
        hbuf_t[j, pl.ds(0, 16)] = one16
        hbuf_t[j, pl.ds(16, 16)] = one16

    for p in range(NPASS):
        @pl.loop(0, rpt // zrows)
        def _(k):
            pltpu.sync_copy(zbuf, acc.at[pl.ds(s * rpt + k * zrows, zrows)])
        plsc.subcore_barrier()

        if p < 4:
            at_s = asrc_tabs[p]
            at_d = adst_tabs[p]

            def body(t, n):
                eb = base + t * CHUNK
                isrc = idx_s if n == CHUNK else idx_st
                idst = idx_d if n == CHUNK else idx_dt
                pltpu.sync_copy(esrc.at[pl.ds(eb, n)], isrc)
                pltpu.sync_copy(edst.at[pl.ds(eb, n)], idst)
                rs = rows_s if n == CHUNK else rows_st
                rd = rows_d if n == CHUNK else rows_dt
                hb = hbuf if n == CHUNK else hbuf_t
                pltpu.sync_copy(at_s.at[isrc], rs)
                pltpu.sync_copy(at_d.at[idst], rd)

                f32 = lambda v: lax.bitcast_convert_type(v, jnp.float32)

                @pl.loop(0, n // 8)
                def _(t8):
                    for u in range(8):
                        j = t8 * 8 + u
                        ws = rs[j, pl.ds(0, 16)]
                        wd = rd[j, pl.ds(0, 16)]
                        he = f32(ws << 16) + f32(wd << 16)
                        ho = f32(ws & -65536) + f32(wd & -65536)
                        hb[j, pl.ds(0, 16)] = jnp.maximum(he, 0.0)
                        hb[j, pl.ds(16, 16)] = jnp.maximum(ho, 0.0)

                pltpu.sync_copy(hb, acc.at[idst], add=True)

            @pl.loop(0, nfull)
            def _(t):
                body(t, CHUNK)

            body(nfull, tail)
            if p == 3:
                @pl.loop(0, CHUNK)
                def _(j):
                    hbuf[j, pl.ds(0, 16)] = one16
                    hbuf[j, pl.ds(16, 16)] = one16

                @pl.loop(0, tail)
                def _(j):
                    hbuf_t[j, pl.ds(0, 16)] = one16
                    hbuf_t[j, pl.ds(16, 16)] = one16
        else:
            @pl.loop(0, nfull)
            def _(t):
                eb = base + t * CHUNK
                pltpu.sync_copy(edst.at[pl.ds(eb, CHUNK)], idx_d)
                pltpu.sync_copy(hbuf, acc.at[idx_d], add=True)

            eb = base + nfull * CHUNK
            pltpu.sync_copy(edst.at[pl.ds(eb, tail)], idx_dt)
            pltpu.sync_copy(hbuf_t, acc.at[idx_dt], add=True)

        plsc.subcore_barrier()
        pltpu.sync_copy(acc.at[pl.ds(s * rpt, rpt)],
                        h_out.at[p, c, pl.ds(s * rpt, rpt)])
        plsc.subcore_barrier()


def _sc_edge(asrc, adst, esrc, edst):
    mesh = plsc.VectorSubcoreMesh(core_axis_name="c", subcore_axis_name="s")
    fn = pl.kernel(
        _sc_edge_kernel,
        out_type=jax.ShapeDtypeStruct((NPASS, 2, N_PAD, 32), jnp.float32),
        mesh=mesh,
        compiler_params=pltpu.CompilerParams(use_tc_tiling_on_sc=False),
        scratch_types=[
            pltpu.VMEM_SHARED((N_PAD, 32), jnp.float32),
            pltpu.VMEM((CHUNK,), jnp.int32),
            pltpu.VMEM((CHUNK,), jnp.int32),
            pltpu.VMEM((40,), jnp.int32),
            pltpu.VMEM((40,), jnp.int32),
            pltpu.VMEM((CHUNK, 16), jnp.int32),
            pltpu.VMEM((CHUNK, 16), jnp.int32),
            pltpu.VMEM((40, 16), jnp.int32),
            pltpu.VMEM((40, 16), jnp.int32),
            pltpu.VMEM((CHUNK, 32), jnp.float32),
            pltpu.VMEM((40, 32), jnp.float32),
            pltpu.VMEM((136, 32), jnp.float32),
        ],
    )
    return fn(asrc[0], asrc[1], asrc[2], asrc[3],
              adst[0], adst[1], adst[2], adst[3], esrc, edst)


def _tc1_body(x_ref, wi1, bi1, wi2, bi2, we1, be1,
              n_ref, as0, as1, as2, as3, ad0, ad1, ad2, ad3):
    x = x_ref[...]
    h = jnp.maximum(jnp.dot(x, wi1[...]) + bi1[...], 0.0)
    n = jnp.dot(h, wi2[...]) + bi2[...]
    n_ref[...] = n
    asrc = jnp.dot(n, we1[0:64, :])
    adst = jnp.dot(n, we1[64:128, :]) + be1[...]

    def pack(chunk):
        b16 = lambda c: lax.bitcast_convert_type(
            c.astype(jnp.bfloat16), jnp.uint16).astype(jnp.uint32)
        lo = b16(chunk[:, 0:16])
        hi = b16(chunk[:, 16:32])
        return lax.bitcast_convert_type(lo | (hi << 16), jnp.int32)

    for p, r in enumerate((as0, as1, as2, as3)):
        r[...] = pack(asrc[:, 32 * p:32 * p + 32])
    for p, r in enumerate((ad0, ad1, ad2, ad3)):
        r[...] = pack(adst[:, 32 * p:32 * p + 32])


def _tc1(x, wi1, bi1, wi2, bi2, we1, be1):
    bn = 1000
    grid = N_NODES // bn
    row_spec = lambda w: pl.BlockSpec((bn, w), lambda ii: (ii, 0))
    full = lambda a: pl.BlockSpec(a.shape, lambda ii: tuple(0 for _ in a.shape))
    out32 = [jax.ShapeDtypeStruct((N_NODES, 16), jnp.int32)] * 8
    return pl.pallas_call(
        _tc1_body,
        grid=(grid,),
        in_specs=[row_spec(64), full(wi1), full(bi1), full(wi2), full(bi2),
                  full(we1), full(be1)],
        out_specs=[row_spec(64)] + [row_spec(16)] * 8,
        out_shape=[jax.ShapeDtypeStruct((N_NODES, 64), jnp.float32)] + out32,
    )(x, wi1, bi1, wi2, bi2, we1, be1)


def _tc2_body(h_ref, n_ref, we2, be2, wv1, bv1, wv2, bv2, wl1, bl1, wl2, bl2,
              out_ref):
    hb = h_ref[...]
    agg = jnp.dot(hb[0, 0] + hb[0, 1], we2[0:32, :])
    for p in range(1, 4):
        agg = agg + jnp.dot(hb[p, 0] + hb[p, 1], we2[32 * p:32 * p + 32, :])
    cnt = hb[4, 0, :, 0:1] + hb[4, 1, :, 0:1]
    agg = agg + cnt * be2[...]
    nin = n_ref[...]
    h2 = jnp.maximum(jnp.dot(agg, wv1[0:64, :]) + jnp.dot(nin, wv1[64:128, :])
                     + bv1[...], 0.0)
    nout = jnp.dot(h2, wv2[...]) + bv2[...]
    h3 = jnp.maximum(jnp.dot(nout, wl1[...]) + bl1[...], 0.0)
    out_ref[...] = jnp.dot(h3, wl2[...]) + bl2[...]


def _tc2(H, n_inp, we2, be2, wv1, bv1, wv2, bv2, wl1, bl1, wl2, bl2):
    bn = 1000
    grid = N_NODES // bn
    full = lambda a: pl.BlockSpec(a.shape, lambda ii: tuple(0 for _ in a.shape))
    return pl.pallas_call(
        _tc2_body,
        grid=(grid,),
        in_specs=[pl.BlockSpec((NPASS, 2, bn, 32), lambda ii: (0, 0, ii, 0)),
                  pl.BlockSpec((bn, 64), lambda ii: (ii, 0)),
                  full(we2), full(be2), full(wv1), full(bv1), full(wv2),
                  full(bv2), full(wl1), full(bl1), full(wl2), full(bl2)],
        out_specs=pl.BlockSpec((bn, 16), lambda ii: (ii, 0)),
        out_shape=jax.ShapeDtypeStruct((N_NODES, 16), jnp.float32),
    )(H, n_inp, we2, be2, wv1, bv1, wv2, bv2, wl1, bl1, wl2, bl2)


def kernel(theta, s, i, edge_index, Wi1, bi1, Wi2, bi2, We1, be1, We2, be2,
           Wv1, bv1, Wv2, bv2, Wu1, bu1, Wu2, bu2, Wl1, bl1, Wl2, bl2):
    B, P, A = theta.shape[0], theta.shape[1], theta.shape[2]
    n = B * P * A
    x = jnp.concatenate(
        [theta.reshape(n, -1), s.reshape(n, -1), i.reshape(n, -1),
         jnp.zeros((n, 15), jnp.float32)], axis=1)
    wi1p = jnp.concatenate([Wi1, jnp.zeros((15, Wi1.shape[1]), jnp.float32)],
                           axis=0)
    r2 = lambda b: b.reshape(1, -1)
    n_inp, as0, as1, as2, as3, ad0, ad1, ad2, ad3 = _tc1(
        x, wi1p, r2(bi1), Wi2, r2(bi2), We1, r2(be1))
    ei32 = edge_index.astype(jnp.int32)
    H = _sc_edge((as0, as1, as2, as3), (ad0, ad1, ad2, ad3),
                 ei32[0], ei32[1])
    out = _tc2(H, n_inp, We2, r2(be2), Wv1, r2(bv1), Wv2, r2(bv2),
               Wl1, r2(bl1), Wl2, r2(bl2))
    return out.reshape(B, P, A, -1)


# speedup vs baseline: 1.7184x; 1.7184x over previous
import functools

import jax
import jax.numpy as jnp
from jax import lax
from jax.experimental import pallas as pl
from jax.experimental.pallas import tpu as pltpu
from jax.experimental.pallas import tpu_sc as plsc

N_NODES = 50000
N_PAD = 50048
N_EDGES = 800000
NPASS = 5
CHUNK = 128


CPW = 196
N_EPAD = 32 * CPW * CHUNK


def _sc_edge_kernel(asrc0, asrc1, asrc2, asrc3, adst0, adst1, adst2, adst3,
                    esrc, edst, h_out, acc, is2, id2,
                    rs0, rs1, rd0, rd1, hbuf, ones, zbuf,
                    semg0, semg1, semi0, semi1):
    c = lax.axis_index("c")
    s = lax.axis_index("s")
    asrc_tabs = (asrc0, asrc1, asrc2, asrc3)
    adst_tabs = (adst0, adst1, adst2, adst3)
    rs = (rs0, rs1)
    rd = (rd0, rd1)
    semg = (semg0, semg1)
    semi = (semi0, semi1)

    wb = (c * 16 + s) * CPW

    rpt = N_PAD // 16
    zrows = 136

    zero16 = jnp.zeros((16,), jnp.float32)
    one16 = jnp.ones((16,), jnp.float32)

    @pl.loop(0, zrows)
    def _(j):
        zbuf[j, pl.ds(0, 16)] = zero16
        zbuf[j, pl.ds(16, 16)] = zero16

    @pl.loop(0, CHUNK)
    def _(j):
        ones[j, pl.ds(0, 16)] = one16
        ones[j, pl.ds(16, 16)] = one16

    def fire_idx(t, b):
        pltpu.async_copy(esrc.at[wb + t], is2.at[b], semi[b])
        pltpu.async_copy(edst.at[wb + t], id2.at[b], semi[b])

    def wait_idx(b):
        pltpu.make_async_copy(esrc.at[0], is2.at[b], semi[b]).wait()
        pltpu.make_async_copy(esrc.at[0], id2.at[b], semi[b]).wait()

    def fire_gat(p, b):
        pltpu.async_copy(asrc_tabs[p].at[is2.at[b]], rs[b], semg[b])
        pltpu.async_copy(adst_tabs[p].at[id2.at[b]], rd[b], semg[b])

    def proc(p, b):
        pltpu.make_async_copy(asrc_tabs[p].at[pl.ds(0, CHUNK)],
                              rs[b], semg[b]).wait()
        pltpu.make_async_copy(asrc_tabs[p].at[pl.ds(0, CHUNK)],
                              rd[b], semg[b]).wait()

        f32 = lambda v: lax.bitcast_convert_type(v, jnp.float32)

        @pl.loop(0, CHUNK // 8)
        def _(t8):
            for u in range(8):
                j = t8 * 8 + u
                ws = rs[b][j, pl.ds(0, 16)]
                wd = rd[b][j, pl.ds(0, 16)]
                he = f32(ws << 16) + f32(wd << 16)
                ho = f32(ws & -65536) + f32(wd & -65536)
                hbuf[j, pl.ds(0, 16)] = jnp.maximum(he, 0.0)
                hbuf[j, pl.ds(16, 16)] = jnp.maximum(ho, 0.0)

        pltpu.sync_copy(hbuf, acc.at[id2.at[b]], add=True)

    for p in range(NPASS):
        @pl.loop(0, rpt // zrows)
        def _(k):
            pltpu.sync_copy(zbuf, acc.at[pl.ds(s * rpt + k * zrows, zrows)])
        plsc.subcore_barrier()

        if p < 4:
            pltpu.sync_copy(esrc.at[wb], is2.at[0])
            pltpu.sync_copy(edst.at[wb], id2.at[0])
            fire_gat(p, 0)
            fire_idx(1, 1)

            def body(t, b):
                wait_idx(1 - b)
                fire_gat(p, 1 - b)
                proc(p, b)
                fire_idx(t + 2, b)

            @pl.loop(0, (CPW - 2) // 2)
            def _(g):
                body(2 * g, 0)
                body(2 * g + 1, 1)

            wait_idx(1)
            fire_gat(p, 1)
            proc(p, 0)
            proc(p, 1)
        else:
            fire_idx(0, 0)

            def body_c(t, b):
                fire_idx(t + 1, 1 - b)
                wait_idx(b)
                pltpu.sync_copy(ones, acc.at[id2.at[b]], add=True)

            @pl.loop(0, (CPW - 2) // 2)
            def _(g):
                body_c(2 * g, 0)
                body_c(2 * g + 1, 1)

            fire_idx(CPW - 1, 1)
            wait_idx(0)
            pltpu.sync_copy(ones, acc.at[id2.at[0]], add=True)
            wait_idx(1)
            pltpu.sync_copy(ones, acc.at[id2.at[1]], add=True)

        plsc.subcore_barrier()
        pltpu.sync_copy(acc.at[pl.ds(s * rpt, rpt)],
                        h_out.at[p, c, pl.ds(s * rpt, rpt)])
        plsc.subcore_barrier()


def _sc_edge(asrc, adst, esrc, edst):
    mesh = plsc.VectorSubcoreMesh(core_axis_name="c", subcore_axis_name="s")
    fn = pl.kernel(
        _sc_edge_kernel,
        out_type=jax.ShapeDtypeStruct((NPASS, 2, N_PAD, 32), jnp.float32),
        mesh=mesh,
        compiler_params=pltpu.CompilerParams(use_tc_tiling_on_sc=False),
        scratch_types=[
            pltpu.VMEM_SHARED((N_PAD, 32), jnp.float32),
            pltpu.VMEM((2, CHUNK), jnp.int32),
            pltpu.VMEM((2, CHUNK), jnp.int32),
            pltpu.VMEM((CHUNK, 16), jnp.int32),
            pltpu.VMEM((CHUNK, 16), jnp.int32),
            pltpu.VMEM((CHUNK, 16), jnp.int32),
            pltpu.VMEM((CHUNK, 16), jnp.int32),
            pltpu.VMEM((CHUNK, 32), jnp.float32),
            pltpu.VMEM((CHUNK, 32), jnp.float32),
            pltpu.VMEM((136, 32), jnp.float32),
            pltpu.SemaphoreType.DMA,
            pltpu.SemaphoreType.DMA,
            pltpu.SemaphoreType.DMA,
            pltpu.SemaphoreType.DMA,
        ],
    )
    return fn(asrc[0], asrc[1], asrc[2], asrc[3],
              adst[0], adst[1], adst[2], adst[3], esrc, edst)


def _tc1_body(x_ref, wi1, bi1, wi2, bi2, we1, be1,
              n_ref, as0, as1, as2, as3, ad0, ad1, ad2, ad3):
    x = x_ref[...]
    h = jnp.maximum(jnp.dot(x, wi1[...]) + bi1[...], 0.0)
    n = jnp.dot(h, wi2[...]) + bi2[...]
    n_ref[...] = n
    asrc = jnp.dot(n, we1[0:64, :])
    adst = jnp.dot(n, we1[64:128, :]) + be1[...]

    def pack(chunk):
        b16 = lambda c: lax.bitcast_convert_type(
            c.astype(jnp.bfloat16), jnp.uint16).astype(jnp.uint32)
        lo = b16(chunk[:, 0:16])
        hi = b16(chunk[:, 16:32])
        return lax.bitcast_convert_type(lo | (hi << 16), jnp.int32)

    for p, r in enumerate((as0, as1, as2, as3)):
        r[...] = pack(asrc[:, 32 * p:32 * p + 32])
    for p, r in enumerate((ad0, ad1, ad2, ad3)):
        r[...] = pack(adst[:, 32 * p:32 * p + 32])


def _tc1(x, wi1, bi1, wi2, bi2, we1, be1):
    bn = 1000
    grid = N_NODES // bn
    row_spec = lambda w: pl.BlockSpec((bn, w), lambda ii: (ii, 0))
    full = lambda a: pl.BlockSpec(a.shape, lambda ii: tuple(0 for _ in a.shape))
    out32 = [jax.ShapeDtypeStruct((N_NODES, 16), jnp.int32)] * 8
    return pl.pallas_call(
        _tc1_body,
        grid=(grid,),
        in_specs=[row_spec(64), full(wi1), full(bi1), full(wi2), full(bi2),
                  full(we1), full(be1)],
        out_specs=[row_spec(64)] + [row_spec(16)] * 8,
        out_shape=[jax.ShapeDtypeStruct((N_NODES, 64), jnp.float32)] + out32,
    )(x, wi1, bi1, wi2, bi2, we1, be1)


def _tc2_body(h_ref, n_ref, we2, be2, wv1, bv1, wv2, bv2, wl1, bl1, wl2, bl2,
              out_ref):
    hb = h_ref[...]
    agg = jnp.dot(hb[0, 0] + hb[0, 1], we2[0:32, :])
    for p in range(1, 4):
        agg = agg + jnp.dot(hb[p, 0] + hb[p, 1], we2[32 * p:32 * p + 32, :])
    cnt = hb[4, 0, :, 0:1] + hb[4, 1, :, 0:1]
    agg = agg + cnt * be2[...]
    nin = n_ref[...]
    h2 = jnp.maximum(jnp.dot(agg, wv1[0:64, :]) + jnp.dot(nin, wv1[64:128, :])
                     + bv1[...], 0.0)
    nout = jnp.dot(h2, wv2[...]) + bv2[...]
    h3 = jnp.maximum(jnp.dot(nout, wl1[...]) + bl1[...], 0.0)
    out_ref[...] = jnp.dot(h3, wl2[...]) + bl2[...]


def _tc2(H, n_inp, we2, be2, wv1, bv1, wv2, bv2, wl1, bl1, wl2, bl2):
    bn = 1000
    grid = N_NODES // bn
    full = lambda a: pl.BlockSpec(a.shape, lambda ii: tuple(0 for _ in a.shape))
    return pl.pallas_call(
        _tc2_body,
        grid=(grid,),
        in_specs=[pl.BlockSpec((NPASS, 2, bn, 32), lambda ii: (0, 0, ii, 0)),
                  pl.BlockSpec((bn, 64), lambda ii: (ii, 0)),
                  full(we2), full(be2), full(wv1), full(bv1), full(wv2),
                  full(bv2), full(wl1), full(bl1), full(wl2), full(bl2)],
        out_specs=pl.BlockSpec((bn, 16), lambda ii: (ii, 0)),
        out_shape=jax.ShapeDtypeStruct((N_NODES, 16), jnp.float32),
    )(H, n_inp, we2, be2, wv1, bv1, wv2, bv2, wl1, bl1, wl2, bl2)


def kernel(theta, s, i, edge_index, Wi1, bi1, Wi2, bi2, We1, be1, We2, be2,
           Wv1, bv1, Wv2, bv2, Wu1, bu1, Wu2, bu2, Wl1, bl1, Wl2, bl2):
    B, P, A = theta.shape[0], theta.shape[1], theta.shape[2]
    n = B * P * A
    x = jnp.concatenate(
        [theta.reshape(n, -1), s.reshape(n, -1), i.reshape(n, -1),
         jnp.zeros((n, 15), jnp.float32)], axis=1)
    wi1p = jnp.concatenate([Wi1, jnp.zeros((15, Wi1.shape[1]), jnp.float32)],
                           axis=0)
    r2 = lambda b: b.reshape(1, -1)
    n_inp, as0, as1, as2, as3, ad0, ad1, ad2, ad3 = _tc1(
        x, wi1p, r2(bi1), Wi2, r2(bi2), We1, r2(be1))
    ei32 = edge_index.astype(jnp.int32)
    npad = N_EPAD - N_EDGES
    esrc2 = jnp.concatenate(
        [ei32[0], jnp.zeros((npad,), jnp.int32)]).reshape(-1, CHUNK)
    edst2 = jnp.concatenate(
        [ei32[1], jnp.full((npad,), N_NODES, jnp.int32)]).reshape(-1, CHUNK)
    H = _sc_edge((as0, as1, as2, as3), (ad0, ad1, ad2, ad3), esrc2, edst2)
    out = _tc2(H, n_inp, We2, r2(be2), Wv1, r2(bv1), Wv2, r2(bv2),
               Wl1, r2(bl1), Wl2, r2(bl2))
    return out.reshape(B, P, A, -1)


# speedup vs baseline: 1.8522x; 1.0779x over previous
import functools

import jax
import jax.numpy as jnp
from jax import lax
from jax.experimental import pallas as pl
from jax.experimental.pallas import tpu as pltpu
from jax.experimental.pallas import tpu_sc as plsc

N_NODES = 50000
N_PAD = 50048
N_EDGES = 800000
NPASS = 5
CHUNK = 128


CPW = 196
N_EPAD = 32 * CPW * CHUNK


def _sc_edge_kernel(asrc0, asrc1, asrc2, asrc3, adst0, adst1, adst2, adst3,
                    esrc, edst, h_out, acc, is2, id2,
                    rs0, rs1, rd0, rd1, hb0, hb1, ones, zbuf,
                    semg0, semg1, semi0, semi1, semh0, semh1):
    c = lax.axis_index("c")
    s = lax.axis_index("s")
    asrc_tabs = (asrc0, asrc1, asrc2, asrc3)
    adst_tabs = (adst0, adst1, adst2, adst3)
    rs = (rs0, rs1)
    rd = (rd0, rd1)
    hb = (hb0, hb1)
    semg = (semg0, semg1)
    semi = (semi0, semi1)
    semh = (semh0, semh1)

    wb = (c * 16 + s) * CPW

    rpt = N_PAD // 16
    zrows = 136

    zero16 = jnp.zeros((16,), jnp.float32)
    one16 = jnp.ones((16,), jnp.float32)

    @pl.loop(0, zrows)
    def _(j):
        zbuf[j, pl.ds(0, 16)] = zero16
        zbuf[j, pl.ds(16, 16)] = zero16

    @pl.loop(0, CHUNK)
    def _(j):
        ones[j, pl.ds(0, 16)] = one16
        ones[j, pl.ds(16, 16)] = one16

    def fire_idx(t, u, b):
        pltpu.async_copy(esrc.at[wb + t], is2.at[u], semi[b])
        pltpu.async_copy(edst.at[wb + t], id2.at[u], semi[b])

    def wait_idx(u, b):
        pltpu.make_async_copy(esrc.at[0], is2.at[u], semi[b]).wait()
        pltpu.make_async_copy(esrc.at[0], id2.at[u], semi[b]).wait()

    def fire_gat(p, u, b):
        pltpu.async_copy(asrc_tabs[p].at[is2.at[u]], rs[b], semg[b])
        pltpu.async_copy(adst_tabs[p].at[id2.at[u]], rd[b], semg[b])

    def wait_gat(p, b):
        pltpu.make_async_copy(asrc_tabs[p].at[pl.ds(0, CHUNK)],
                              rs[b], semg[b]).wait()
        pltpu.make_async_copy(asrc_tabs[p].at[pl.ds(0, CHUNK)],
                              rd[b], semg[b]).wait()

    def drain_h(b):
        pltpu.make_async_copy(h_out.at[0, 0, pl.ds(0, CHUNK)],
                              hb[b], semh[b]).wait()

    def compute(b):
        f32 = lambda v: lax.bitcast_convert_type(v, jnp.float32)

        @pl.loop(0, CHUNK // 8)
        def _(t8):
            for u in range(8):
                j = t8 * 8 + u
                ws = rs[b][j, pl.ds(0, 16)]
                wd = rd[b][j, pl.ds(0, 16)]
                he = f32(ws << 16) + f32(wd << 16)
                ho = f32(ws & -65536) + f32(wd & -65536)
                hb[b][j, pl.ds(0, 16)] = jnp.maximum(he, 0.0)
                hb[b][j, pl.ds(16, 16)] = jnp.maximum(ho, 0.0)

    for p in range(NPASS):
        @pl.loop(0, rpt // zrows)
        def _(k):
            pltpu.sync_copy(zbuf, acc.at[pl.ds(s * rpt + k * zrows, zrows)])
        plsc.subcore_barrier()

        if p < 4:
            def body(t, u, drain):
                b = u % 2
                wait_idx((u + 1) % 4, 1 - b)
                fire_gat(p, (u + 1) % 4, 1 - b)
                wait_gat(p, b)
                if drain:
                    drain_h(b)
                compute(b)
                pltpu.async_copy(hb[b], acc.at[id2.at[u]], semh[b], add=True)
                fire_idx(t + 2, (u + 2) % 4, b)

            pltpu.sync_copy(esrc.at[wb], is2.at[0])
            pltpu.sync_copy(edst.at[wb], id2.at[0])
            fire_gat(p, 0, 0)
            fire_idx(1, 1, 1)
            body(0, 0, False)
            body(1, 1, False)

            @pl.loop(0, (CPW - 4) // 4)
            def _(q):
                t0 = 2 + 4 * q
                body(t0, 2, True)
                body(t0 + 1, 3, True)
                body(t0 + 2, 0, True)
                body(t0 + 3, 1, True)

            wait_idx(3, 1)
            fire_gat(p, 3, 1)
            wait_gat(p, 0)
            drain_h(0)
            compute(0)
            pltpu.async_copy(hb[0], acc.at[id2.at[2]], semh[0], add=True)
            wait_gat(p, 1)
            drain_h(1)
            compute(1)
            pltpu.async_copy(hb[1], acc.at[id2.at[3]], semh[1], add=True)
            drain_h(0)
            drain_h(1)
        else:
            fire_idx(0, 0, 0)

            def body_c(t, b):
                fire_idx(t + 1, 1 - b, 1 - b)
                wait_idx(b, b)
                pltpu.sync_copy(ones, acc.at[id2.at[b]], add=True)

            @pl.loop(0, (CPW - 2) // 2)
            def _(g):
                body_c(2 * g, 0)
                body_c(2 * g + 1, 1)

            fire_idx(CPW - 1, 1, 1)
            wait_idx(0, 0)
            pltpu.sync_copy(ones, acc.at[id2.at[0]], add=True)
            wait_idx(1, 1)
            pltpu.sync_copy(ones, acc.at[id2.at[1]], add=True)

        plsc.subcore_barrier()
        pltpu.sync_copy(acc.at[pl.ds(s * rpt, rpt)],
                        h_out.at[p, c, pl.ds(s * rpt, rpt)])
        plsc.subcore_barrier()


def _sc_edge(asrc, adst, esrc, edst):
    mesh = plsc.VectorSubcoreMesh(core_axis_name="c", subcore_axis_name="s")
    fn = pl.kernel(
        _sc_edge_kernel,
        out_type=jax.ShapeDtypeStruct((NPASS, 2, N_PAD, 32), jnp.float32),
        mesh=mesh,
        compiler_params=pltpu.CompilerParams(use_tc_tiling_on_sc=False),
        scratch_types=[
            pltpu.VMEM_SHARED((N_PAD, 32), jnp.float32),
            pltpu.VMEM((4, CHUNK), jnp.int32),
            pltpu.VMEM((4, CHUNK), jnp.int32),
            pltpu.VMEM((CHUNK, 16), jnp.int32),
            pltpu.VMEM((CHUNK, 16), jnp.int32),
            pltpu.VMEM((CHUNK, 16), jnp.int32),
            pltpu.VMEM((CHUNK, 16), jnp.int32),
            pltpu.VMEM((CHUNK, 32), jnp.float32),
            pltpu.VMEM((CHUNK, 32), jnp.float32),
            pltpu.VMEM((CHUNK, 32), jnp.float32),
            pltpu.VMEM((136, 32), jnp.float32),
            pltpu.SemaphoreType.DMA,
            pltpu.SemaphoreType.DMA,
            pltpu.SemaphoreType.DMA,
            pltpu.SemaphoreType.DMA,
            pltpu.SemaphoreType.DMA,
            pltpu.SemaphoreType.DMA,
        ],
    )
    return fn(asrc[0], asrc[1], asrc[2], asrc[3],
              adst[0], adst[1], adst[2], adst[3], esrc, edst)


def _tc1_body(x_ref, wi1, bi1, wi2, bi2, we1, be1,
              n_ref, as0, as1, as2, as3, ad0, ad1, ad2, ad3):
    x = x_ref[...]
    h = jnp.maximum(jnp.dot(x, wi1[...]) + bi1[...], 0.0)
    n = jnp.dot(h, wi2[...]) + bi2[...]
    n_ref[...] = n
    asrc = jnp.dot(n, we1[0:64, :])
    adst = jnp.dot(n, we1[64:128, :]) + be1[...]

    def pack(chunk):
        b16 = lambda c: lax.bitcast_convert_type(
            c.astype(jnp.bfloat16), jnp.uint16).astype(jnp.uint32)
        lo = b16(chunk[:, 0:16])
        hi = b16(chunk[:, 16:32])
        return lax.bitcast_convert_type(lo | (hi << 16), jnp.int32)

    for p, r in enumerate((as0, as1, as2, as3)):
        r[...] = pack(asrc[:, 32 * p:32 * p + 32])
    for p, r in enumerate((ad0, ad1, ad2, ad3)):
        r[...] = pack(adst[:, 32 * p:32 * p + 32])


def _tc1(x, wi1, bi1, wi2, bi2, we1, be1):
    bn = 1000
    grid = N_NODES // bn
    row_spec = lambda w: pl.BlockSpec((bn, w), lambda ii: (ii, 0))
    full = lambda a: pl.BlockSpec(a.shape, lambda ii: tuple(0 for _ in a.shape))
    out32 = [jax.ShapeDtypeStruct((N_NODES, 16), jnp.int32)] * 8
    return pl.pallas_call(
        _tc1_body,
        grid=(grid,),
        in_specs=[row_spec(64), full(wi1), full(bi1), full(wi2), full(bi2),
                  full(we1), full(be1)],
        out_specs=[row_spec(64)] + [row_spec(16)] * 8,
        out_shape=[jax.ShapeDtypeStruct((N_NODES, 64), jnp.float32)] + out32,
    )(x, wi1, bi1, wi2, bi2, we1, be1)


def _tc2_body(h_ref, n_ref, we2, be2, wv1, bv1, wv2, bv2, wl1, bl1, wl2, bl2,
              out_ref):
    hb = h_ref[...]
    agg = jnp.dot(hb[0, 0] + hb[0, 1], we2[0:32, :])
    for p in range(1, 4):
        agg = agg + jnp.dot(hb[p, 0] + hb[p, 1], we2[32 * p:32 * p + 32, :])
    cnt = hb[4, 0, :, 0:1] + hb[4, 1, :, 0:1]
    agg = agg + cnt * be2[...]
    nin = n_ref[...]
    h2 = jnp.maximum(jnp.dot(agg, wv1[0:64, :]) + jnp.dot(nin, wv1[64:128, :])
                     + bv1[...], 0.0)
    nout = jnp.dot(h2, wv2[...]) + bv2[...]
    h3 = jnp.maximum(jnp.dot(nout, wl1[...]) + bl1[...], 0.0)
    out_ref[...] = jnp.dot(h3, wl2[...]) + bl2[...]


def _tc2(H, n_inp, we2, be2, wv1, bv1, wv2, bv2, wl1, bl1, wl2, bl2):
    bn = 1000
    grid = N_NODES // bn
    full = lambda a: pl.BlockSpec(a.shape, lambda ii: tuple(0 for _ in a.shape))
    return pl.pallas_call(
        _tc2_body,
        grid=(grid,),
        in_specs=[pl.BlockSpec((NPASS, 2, bn, 32), lambda ii: (0, 0, ii, 0)),
                  pl.BlockSpec((bn, 64), lambda ii: (ii, 0)),
                  full(we2), full(be2), full(wv1), full(bv1), full(wv2),
                  full(bv2), full(wl1), full(bl1), full(wl2), full(bl2)],
        out_specs=pl.BlockSpec((bn, 16), lambda ii: (ii, 0)),
        out_shape=jax.ShapeDtypeStruct((N_NODES, 16), jnp.float32),
    )(H, n_inp, we2, be2, wv1, bv1, wv2, bv2, wl1, bl1, wl2, bl2)


def kernel(theta, s, i, edge_index, Wi1, bi1, Wi2, bi2, We1, be1, We2, be2,
           Wv1, bv1, Wv2, bv2, Wu1, bu1, Wu2, bu2, Wl1, bl1, Wl2, bl2):
    B, P, A = theta.shape[0], theta.shape[1], theta.shape[2]
    n = B * P * A
    x = jnp.concatenate(
        [theta.reshape(n, -1), s.reshape(n, -1), i.reshape(n, -1),
         jnp.zeros((n, 15), jnp.float32)], axis=1)
    wi1p = jnp.concatenate([Wi1, jnp.zeros((15, Wi1.shape[1]), jnp.float32)],
                           axis=0)
    r2 = lambda b: b.reshape(1, -1)
    n_inp, as0, as1, as2, as3, ad0, ad1, ad2, ad3 = _tc1(
        x, wi1p, r2(bi1), Wi2, r2(bi2), We1, r2(be1))
    ei32 = edge_index.astype(jnp.int32)
    npad = N_EPAD - N_EDGES
    esrc2 = jnp.concatenate(
        [ei32[0], jnp.zeros((npad,), jnp.int32)]).reshape(-1, CHUNK)
    edst2 = jnp.concatenate(
        [ei32[1], jnp.full((npad,), N_NODES, jnp.int32)]).reshape(-1, CHUNK)
    H = _sc_edge((as0, as1, as2, as3), (ad0, ad1, ad2, ad3), esrc2, edst2)
    out = _tc2(H, n_inp, We2, r2(be2), Wv1, r2(bv1), Wv2, r2(bv2),
               Wl1, r2(bl1), Wl2, r2(bl2))
    return out.reshape(B, P, A, -1)


# speedup vs baseline: 1.8841x; 1.0172x over previous
import functools

import jax
import jax.numpy as jnp
from jax import lax
from jax.experimental import pallas as pl
from jax.experimental.pallas import tpu as pltpu
from jax.experimental.pallas import tpu_sc as plsc

N_NODES = 50000
N_PAD = 50048
N_EDGES = 800000
NPASS = 5
CHUNK = 128


CPW = 196
N_EPAD = 32 * CPW * CHUNK


def _sc_edge_kernel(asrc0, asrc1, asrc2, asrc3, adst0, adst1, adst2, adst3,
                    esrc, edst, h_out, acc, is2, id2,
                    rs0, rs1, rd0, rd1, hb0, hb1, ones, zbuf,
                    semg0, semg1, semi0, semi1, semh0, semh1):
    c = lax.axis_index("c")
    s = lax.axis_index("s")
    asrc_tabs = (asrc0, asrc1, asrc2, asrc3)
    adst_tabs = (adst0, adst1, adst2, adst3)
    rs = (rs0, rs1)
    rd = (rd0, rd1)
    hb = (hb0, hb1)
    semg = (semg0, semg1)
    semi = (semi0, semi1)
    semh = (semh0, semh1)

    wb = (c * 16 + s) * CPW

    rpt = N_PAD // 16
    zrows = 136

    zero16 = jnp.zeros((16,), jnp.float32)
    one16 = jnp.ones((16,), jnp.float32)

    @pl.loop(0, zrows)
    def _(j):
        zbuf[j, pl.ds(0, 16)] = zero16
        zbuf[j, pl.ds(16, 16)] = zero16

    @pl.loop(0, CHUNK)
    def _(j):
        ones[j, pl.ds(0, 16)] = one16
        ones[j, pl.ds(16, 16)] = one16

    def fire_idx(t, u, b):
        pltpu.async_copy(esrc.at[wb + t], is2.at[u], semi[b])
        pltpu.async_copy(edst.at[wb + t], id2.at[u], semi[b])

    def wait_idx(u, b):
        pltpu.make_async_copy(esrc.at[0], is2.at[u], semi[b]).wait()
        pltpu.make_async_copy(esrc.at[0], id2.at[u], semi[b]).wait()

    def fire_gat(p, u, b):
        pltpu.async_copy(asrc_tabs[p].at[is2.at[u]], rs[b], semg[b])
        pltpu.async_copy(adst_tabs[p].at[id2.at[u]], rd[b], semg[b])

    def wait_gat(p, b):
        pltpu.make_async_copy(asrc_tabs[p].at[pl.ds(0, CHUNK)],
                              rs[b], semg[b]).wait()
        pltpu.make_async_copy(asrc_tabs[p].at[pl.ds(0, CHUNK)],
                              rd[b], semg[b]).wait()

    def drain_h(b):
        pltpu.make_async_copy(h_out.at[0, 0, pl.ds(0, CHUNK)],
                              hb[b], semh[b]).wait()

    def compute(b):
        f32 = lambda v: lax.bitcast_convert_type(v, jnp.float32)

        @pl.loop(0, CHUNK // 8)
        def _(t8):
            for u in range(8):
                j = t8 * 8 + u
                ws = rs[b][j, pl.ds(0, 16)]
                wd = rd[b][j, pl.ds(0, 16)]
                he = f32(ws << 16) + f32(wd << 16)
                ho = f32(ws & -65536) + f32(wd & -65536)
                hb[b][j, pl.ds(0, 16)] = jnp.maximum(he, 0.0)
                hb[b][j, pl.ds(16, 16)] = jnp.maximum(ho, 0.0)

    for p in range(NPASS):
        @pl.loop(0, rpt // zrows)
        def _(k):
            pltpu.sync_copy(zbuf, acc.at[pl.ds(s * rpt + k * zrows, zrows)])
        plsc.subcore_barrier()

        if p < 4:
            def body(t, u, drain):
                b = u % 2
                wait_idx((u + 1) % 4, 1 - b)
                fire_gat(p, (u + 1) % 4, 1 - b)
                wait_gat(p, b)
                if drain:
                    drain_h(b)
                compute(b)
                pltpu.async_copy(hb[b], acc.at[id2.at[u]], semh[b], add=True)
                fire_idx(t + 2, (u + 2) % 4, b)

            pltpu.sync_copy(esrc.at[wb], is2.at[0])
            pltpu.sync_copy(edst.at[wb], id2.at[0])
            fire_gat(p, 0, 0)
            fire_idx(1, 1, 1)
            body(0, 0, False)
            body(1, 1, False)

            @pl.loop(0, (CPW - 4) // 4)
            def _(q):
                t0 = 2 + 4 * q
                body(t0, 2, True)
                body(t0 + 1, 3, True)
                body(t0 + 2, 0, True)
                body(t0 + 3, 1, True)

            wait_idx(3, 1)
            fire_gat(p, 3, 1)
            wait_gat(p, 0)
            drain_h(0)
            compute(0)
            pltpu.async_copy(hb[0], acc.at[id2.at[2]], semh[0], add=True)
            wait_gat(p, 1)
            drain_h(1)
            compute(1)
            pltpu.async_copy(hb[1], acc.at[id2.at[3]], semh[1], add=True)
            drain_h(0)
            drain_h(1)
        else:
            fire_idx(0, 0, 0)

            def body_c(t, b):
                fire_idx(t + 1, 1 - b, 1 - b)
                wait_idx(b, b)
                pltpu.sync_copy(ones, acc.at[id2.at[b]], add=True)

            @pl.loop(0, (CPW - 2) // 2)
            def _(g):
                body_c(2 * g, 0)
                body_c(2 * g + 1, 1)

            fire_idx(CPW - 1, 1, 1)
            wait_idx(0, 0)
            pltpu.sync_copy(ones, acc.at[id2.at[0]], add=True)
            wait_idx(1, 1)
            pltpu.sync_copy(ones, acc.at[id2.at[1]], add=True)

        plsc.subcore_barrier()
        pltpu.sync_copy(acc.at[pl.ds(s * rpt, rpt)],
                        h_out.at[p, c, pl.ds(s * rpt, rpt)])
        plsc.subcore_barrier()


def _sc_edge(asrc, adst, esrc, edst):
    mesh = plsc.VectorSubcoreMesh(core_axis_name="c", subcore_axis_name="s")
    fn = pl.kernel(
        _sc_edge_kernel,
        out_type=jax.ShapeDtypeStruct((NPASS, 2, N_PAD, 32), jnp.float32),
        mesh=mesh,
        compiler_params=pltpu.CompilerParams(use_tc_tiling_on_sc=False),
        scratch_types=[
            pltpu.VMEM_SHARED((N_PAD, 32), jnp.float32),
            pltpu.VMEM((4, CHUNK), jnp.int32),
            pltpu.VMEM((4, CHUNK), jnp.int32),
            pltpu.VMEM((CHUNK, 16), jnp.int32),
            pltpu.VMEM((CHUNK, 16), jnp.int32),
            pltpu.VMEM((CHUNK, 16), jnp.int32),
            pltpu.VMEM((CHUNK, 16), jnp.int32),
            pltpu.VMEM((CHUNK, 32), jnp.float32),
            pltpu.VMEM((CHUNK, 32), jnp.float32),
            pltpu.VMEM((CHUNK, 32), jnp.float32),
            pltpu.VMEM((136, 32), jnp.float32),
            pltpu.SemaphoreType.DMA,
            pltpu.SemaphoreType.DMA,
            pltpu.SemaphoreType.DMA,
            pltpu.SemaphoreType.DMA,
            pltpu.SemaphoreType.DMA,
            pltpu.SemaphoreType.DMA,
        ],
    )
    return fn(asrc[0], asrc[1], asrc[2], asrc[3],
              adst[0], adst[1], adst[2], adst[3], esrc, edst)


def _tc1_body(x_ref, wi1, bi1, wi2, bi2, we1, be1,
              n_ref, as0, as1, as2, as3, ad0, ad1, ad2, ad3):
    x = x_ref[...]
    h = jnp.maximum(jnp.dot(x, wi1[...]) + bi1[...], 0.0)
    n = jnp.dot(h, wi2[...]) + bi2[...]
    n_ref[...] = n
    asrc = jnp.dot(n, we1[0:64, :])
    adst = jnp.dot(n, we1[64:128, :]) + be1[...]

    def pack(chunk):
        b16 = lambda c: lax.bitcast_convert_type(
            c.astype(jnp.bfloat16), jnp.uint16).astype(jnp.uint32)
        lo = b16(chunk[:, 0:16])
        hi = b16(chunk[:, 16:32])
        return lax.bitcast_convert_type(lo | (hi << 16), jnp.int32)

    for p, r in enumerate((as0, as1, as2, as3)):
        r[...] = pack(asrc[:, 32 * p:32 * p + 32])
    for p, r in enumerate((ad0, ad1, ad2, ad3)):
        r[...] = pack(adst[:, 32 * p:32 * p + 32])


def _tc1(x, wi1, bi1, wi2, bi2, we1, be1):
    bn = 2000
    grid = N_NODES // bn
    row_spec = lambda w: pl.BlockSpec((bn, w), lambda ii: (ii, 0))
    full = lambda a: pl.BlockSpec(a.shape, lambda ii: tuple(0 for _ in a.shape))
    out32 = [jax.ShapeDtypeStruct((N_NODES, 16), jnp.int32)] * 8
    return pl.pallas_call(
        _tc1_body,
        grid=(grid,),
        in_specs=[row_spec(64), full(wi1), full(bi1), full(wi2), full(bi2),
                  full(we1), full(be1)],
        out_specs=[row_spec(64)] + [row_spec(16)] * 8,
        out_shape=[jax.ShapeDtypeStruct((N_NODES, 64), jnp.float32)] + out32,
    )(x, wi1, bi1, wi2, bi2, we1, be1)


def _tc2_body(h_ref, n_ref, we2, be2, wv1, bv1, wv2, bv2, wl1, bl1, wl2, bl2,
              out_ref):
    hb = h_ref[...]
    agg = jnp.dot(hb[0, 0] + hb[0, 1], we2[0:32, :])
    for p in range(1, 4):
        agg = agg + jnp.dot(hb[p, 0] + hb[p, 1], we2[32 * p:32 * p + 32, :])
    cnt = hb[4, 0, :, 0:1] + hb[4, 1, :, 0:1]
    agg = agg + cnt * be2[...]
    nin = n_ref[...]
    h2 = jnp.maximum(jnp.dot(agg, wv1[0:64, :]) + jnp.dot(nin, wv1[64:128, :])
                     + bv1[...], 0.0)
    nout = jnp.dot(h2, wv2[...]) + bv2[...]
    h3 = jnp.maximum(jnp.dot(nout, wl1[...]) + bl1[...], 0.0)
    out_ref[...] = jnp.dot(h3, wl2[...]) + bl2[...]


def _tc2(H, n_inp, we2, be2, wv1, bv1, wv2, bv2, wl1, bl1, wl2, bl2):
    bn = 2000
    grid = N_NODES // bn
    full = lambda a: pl.BlockSpec(a.shape, lambda ii: tuple(0 for _ in a.shape))
    return pl.pallas_call(
        _tc2_body,
        grid=(grid,),
        in_specs=[pl.BlockSpec((NPASS, 2, bn, 32), lambda ii: (0, 0, ii, 0)),
                  pl.BlockSpec((bn, 64), lambda ii: (ii, 0)),
                  full(we2), full(be2), full(wv1), full(bv1), full(wv2),
                  full(bv2), full(wl1), full(bl1), full(wl2), full(bl2)],
        out_specs=pl.BlockSpec((bn, 16), lambda ii: (ii, 0)),
        out_shape=jax.ShapeDtypeStruct((N_NODES, 16), jnp.float32),
    )(H, n_inp, we2, be2, wv1, bv1, wv2, bv2, wl1, bl1, wl2, bl2)


def kernel(theta, s, i, edge_index, Wi1, bi1, Wi2, bi2, We1, be1, We2, be2,
           Wv1, bv1, Wv2, bv2, Wu1, bu1, Wu2, bu2, Wl1, bl1, Wl2, bl2):
    B, P, A = theta.shape[0], theta.shape[1], theta.shape[2]
    n = B * P * A
    x = jnp.concatenate(
        [theta.reshape(n, -1), s.reshape(n, -1), i.reshape(n, -1),
         jnp.zeros((n, 15), jnp.float32)], axis=1)
    wi1p = jnp.concatenate([Wi1, jnp.zeros((15, Wi1.shape[1]), jnp.float32)],
                           axis=0)
    r2 = lambda b: b.reshape(1, -1)
    n_inp, as0, as1, as2, as3, ad0, ad1, ad2, ad3 = _tc1(
        x, wi1p, r2(bi1), Wi2, r2(bi2), We1, r2(be1))
    ei32 = edge_index.astype(jnp.int32)
    npad = N_EPAD - N_EDGES
    esrc2 = jnp.concatenate(
        [ei32[0], jnp.zeros((npad,), jnp.int32)]).reshape(-1, CHUNK)
    edst2 = jnp.concatenate(
        [ei32[1], jnp.full((npad,), N_NODES, jnp.int32)]).reshape(-1, CHUNK)
    H = _sc_edge((as0, as1, as2, as3), (ad0, ad1, ad2, ad3), esrc2, edst2)
    out = _tc2(H, n_inp, We2, r2(be2), Wv1, r2(bv1), Wv2, r2(bv2),
               Wl1, r2(bl1), Wl2, r2(bl2))
    return out.reshape(B, P, A, -1)


# speedup vs baseline: 2.0006x; 1.0618x over previous
import functools

import jax
import jax.numpy as jnp
from jax import lax
from jax.experimental import pallas as pl
from jax.experimental.pallas import tpu as pltpu
from jax.experimental.pallas import tpu_sc as plsc

N_NODES = 50000
N_PAD = 50048
N_EDGES = 800000
CHUNK = 128


CPW = 196
N_EPAD = 32 * CPW * CHUNK


def _sc_count_kernel(edst, cnt_out, acc, id2, ones, zbuf, semi0, semi1):
    c = lax.axis_index("c")
    s = lax.axis_index("s")
    semi = (semi0, semi1)
    wb = (c * 16 + s) * CPW
    rpt = N_PAD // 16
    zrows = 136

    zero16 = jnp.zeros((16,), jnp.float32)
    one16 = jnp.ones((16,), jnp.float32)

    @pl.loop(0, zrows)
    def _(j):
        zbuf[j, pl.ds(0, 16)] = zero16

    @pl.loop(0, CHUNK)
    def _(j):
        ones[j, pl.ds(0, 16)] = one16

    @pl.loop(0, rpt // zrows)
    def _(k):
        pltpu.sync_copy(zbuf, acc.at[pl.ds(s * rpt + k * zrows, zrows)])
    plsc.subcore_barrier()

    def fire(t, b):
        pltpu.async_copy(edst.at[wb + t], id2.at[b], semi[b])

    def wait(b):
        pltpu.make_async_copy(edst.at[0], id2.at[b], semi[b]).wait()

    fire(0, 0)

    def body_c(t, b):
        fire(t + 1, 1 - b)
        wait(b)
        pltpu.sync_copy(ones, acc.at[id2.at[b]], add=True)

    @pl.loop(0, (CPW - 2) // 2)
    def _(g):
        body_c(2 * g, 0)
        body_c(2 * g + 1, 1)

    fire(CPW - 1, 1)
    wait(0)
    pltpu.sync_copy(ones, acc.at[id2.at[0]], add=True)
    wait(1)
    pltpu.sync_copy(ones, acc.at[id2.at[1]], add=True)

    plsc.subcore_barrier()
    pltpu.sync_copy(acc.at[pl.ds(s * rpt, rpt)],
                    cnt_out.at[c, pl.ds(s * rpt, rpt)])


def _sc_count(edst):
    mesh = plsc.VectorSubcoreMesh(core_axis_name="c", subcore_axis_name="s")
    fn = pl.kernel(
        _sc_count_kernel,
        out_type=jax.ShapeDtypeStruct((2, N_PAD, 16), jnp.float32),
        mesh=mesh,
        compiler_params=pltpu.CompilerParams(use_tc_tiling_on_sc=False),
        scratch_types=[
            pltpu.VMEM_SHARED((N_PAD, 16), jnp.float32),
            pltpu.VMEM((2, CHUNK), jnp.int32),
            pltpu.VMEM((CHUNK, 16), jnp.float32),
            pltpu.VMEM((136, 16), jnp.float32),
            pltpu.SemaphoreType.DMA,
            pltpu.SemaphoreType.DMA,
        ],
    )
    return fn(edst)


def _sc_edge_kernel(asrc0, asrc1, asrc2, asrc3, adst0, adst1, adst2, adst3,
                    esrc, edst, h_out, acc, is2, id2,
                    rs0, rs1, rd0, rd1, hb0, hb1, zbuf,
                    semg0, semg1, semi0, semi1, semh0, semh1):
    c = lax.axis_index("c")
    s = lax.axis_index("s")
    asrc_tabs = (asrc0, asrc1, asrc2, asrc3)
    adst_tabs = (adst0, adst1, adst2, adst3)
    rs = (rs0, rs1)
    rd = (rd0, rd1)
    hb = (hb0, hb1)
    semg = (semg0, semg1)
    semi = (semi0, semi1)
    semh = (semh0, semh1)

    wb = (c * 16 + s) * CPW

    rpt = N_PAD // 16
    zrows = 136

    zero16 = jnp.zeros((16,), jnp.float32)

    @pl.loop(0, zrows)
    def _(j):
        zbuf[j, pl.ds(0, 16)] = zero16
        zbuf[j, pl.ds(16, 16)] = zero16

    def fire_idx(t, u, b):
        pltpu.async_copy(esrc.at[wb + t], is2.at[u], semi[b])
        pltpu.async_copy(edst.at[wb + t], id2.at[u], semi[b])

    def wait_idx(u, b):
        pltpu.make_async_copy(esrc.at[0], is2.at[u], semi[b]).wait()
        pltpu.make_async_copy(esrc.at[0], id2.at[u], semi[b]).wait()

    def fire_gat(p, u, b):
        pltpu.async_copy(asrc_tabs[p].at[is2.at[u]], rs[b], semg[b])
        pltpu.async_copy(adst_tabs[p].at[id2.at[u]], rd[b], semg[b])

    def wait_gat(p, b):
        pltpu.make_async_copy(asrc_tabs[p].at[pl.ds(0, CHUNK)],
                              rs[b], semg[b]).wait()
        pltpu.make_async_copy(asrc_tabs[p].at[pl.ds(0, CHUNK)],
                              rd[b], semg[b]).wait()

    def drain_h(b):
        pltpu.make_async_copy(h_out.at[0, 0, pl.ds(0, CHUNK)],
                              hb[b], semh[b]).wait()

    def compute(b):
        f32 = lambda v: lax.bitcast_convert_type(v, jnp.float32)

        @pl.loop(0, CHUNK // 8)
        def _(t8):
            for u in range(8):
                j = t8 * 8 + u
                ws = rs[b][j, pl.ds(0, 16)]
                wd = rd[b][j, pl.ds(0, 16)]
                he = f32(ws << 16) + f32(wd << 16)
                ho = f32(ws & -65536) + f32(wd & -65536)
                hb[b][j, pl.ds(0, 16)] = jnp.maximum(he, 0.0)
                hb[b][j, pl.ds(16, 16)] = jnp.maximum(ho, 0.0)

    for p in range(4):
        @pl.loop(0, rpt // zrows)
        def _(k):
            pltpu.sync_copy(zbuf, acc.at[pl.ds(s * rpt + k * zrows, zrows)])
        plsc.subcore_barrier()

        def body(t, u, drain):
            b = u % 2
            wait_idx((u + 1) % 4, 1 - b)
            fire_gat(p, (u + 1) % 4, 1 - b)
            wait_gat(p, b)
            if drain:
                drain_h(b)
            compute(b)
            pltpu.async_copy(hb[b], acc.at[id2.at[u]], semh[b], add=True)
            fire_idx(t + 2, (u + 2) % 4, b)

        pltpu.sync_copy(esrc.at[wb], is2.at[0])
        pltpu.sync_copy(edst.at[wb], id2.at[0])
        fire_gat(p, 0, 0)
        fire_idx(1, 1, 1)
        body(0, 0, False)
        body(1, 1, False)

        @pl.loop(0, (CPW - 4) // 4)
        def _(q):
            t0 = 2 + 4 * q
            body(t0, 2, True)
            body(t0 + 1, 3, True)
            body(t0 + 2, 0, True)
            body(t0 + 3, 1, True)

        wait_idx(3, 1)
        fire_gat(p, 3, 1)
        wait_gat(p, 0)
        drain_h(0)
        compute(0)
        pltpu.async_copy(hb[0], acc.at[id2.at[2]], semh[0], add=True)
        wait_gat(p, 1)
        drain_h(1)
        compute(1)
        pltpu.async_copy(hb[1], acc.at[id2.at[3]], semh[1], add=True)
        drain_h(0)
        drain_h(1)

        plsc.subcore_barrier()
        pltpu.sync_copy(acc.at[pl.ds(s * rpt, rpt)],
                        h_out.at[p, c, pl.ds(s * rpt, rpt)])
        plsc.subcore_barrier()


def _sc_edge(asrc, adst, esrc, edst):
    mesh = plsc.VectorSubcoreMesh(core_axis_name="c", subcore_axis_name="s")
    fn = pl.kernel(
        _sc_edge_kernel,
        out_type=jax.ShapeDtypeStruct((4, 2, N_PAD, 32), jnp.float32),
        mesh=mesh,
        compiler_params=pltpu.CompilerParams(use_tc_tiling_on_sc=False),
        scratch_types=[
            pltpu.VMEM_SHARED((N_PAD, 32), jnp.float32),
            pltpu.VMEM((4, CHUNK), jnp.int32),
            pltpu.VMEM((4, CHUNK), jnp.int32),
            pltpu.VMEM((CHUNK, 16), jnp.int32),
            pltpu.VMEM((CHUNK, 16), jnp.int32),
            pltpu.VMEM((CHUNK, 16), jnp.int32),
            pltpu.VMEM((CHUNK, 16), jnp.int32),
            pltpu.VMEM((CHUNK, 32), jnp.float32),
            pltpu.VMEM((CHUNK, 32), jnp.float32),
            pltpu.VMEM((136, 32), jnp.float32),
            pltpu.SemaphoreType.DMA,
            pltpu.SemaphoreType.DMA,
            pltpu.SemaphoreType.DMA,
            pltpu.SemaphoreType.DMA,
            pltpu.SemaphoreType.DMA,
            pltpu.SemaphoreType.DMA,
        ],
    )
    return fn(asrc[0], asrc[1], asrc[2], asrc[3],
              adst[0], adst[1], adst[2], adst[3], esrc, edst)


def _tc1_body(x_ref, wi1, bi1, wi2, bi2, we1, be1,
              n_ref, as0, as1, as2, as3, ad0, ad1, ad2, ad3):
    x = x_ref[...]
    h = jnp.maximum(jnp.dot(x, wi1[...]) + bi1[...], 0.0)
    n = jnp.dot(h, wi2[...]) + bi2[...]
    n_ref[...] = n
    asrc = jnp.dot(n, we1[0:64, :])
    adst = jnp.dot(n, we1[64:128, :]) + be1[...]

    def pack(chunk):
        b16 = lambda c: lax.bitcast_convert_type(
            c.astype(jnp.bfloat16), jnp.uint16).astype(jnp.uint32)
        lo = b16(chunk[:, 0:16])
        hi = b16(chunk[:, 16:32])
        return lax.bitcast_convert_type(lo | (hi << 16), jnp.int32)

    for p, r in enumerate((as0, as1, as2, as3)):
        r[...] = pack(asrc[:, 32 * p:32 * p + 32])
    for p, r in enumerate((ad0, ad1, ad2, ad3)):
        r[...] = pack(adst[:, 32 * p:32 * p + 32])


def _tc1(x, wi1, bi1, wi2, bi2, we1, be1):
    bn = 2000
    grid = N_NODES // bn
    row_spec = lambda w: pl.BlockSpec((bn, w), lambda ii: (ii, 0))
    full = lambda a: pl.BlockSpec(a.shape, lambda ii: tuple(0 for _ in a.shape))
    out32 = [jax.ShapeDtypeStruct((N_NODES, 16), jnp.int32)] * 8
    return pl.pallas_call(
        _tc1_body,
        grid=(grid,),
        in_specs=[row_spec(64), full(wi1), full(bi1), full(wi2), full(bi2),
                  full(we1), full(be1)],
        out_specs=[row_spec(64)] + [row_spec(16)] * 8,
        out_shape=[jax.ShapeDtypeStruct((N_NODES, 64), jnp.float32)] + out32,
    )(x, wi1, bi1, wi2, bi2, we1, be1)


def _tc2_body(h_ref, c_ref, n_ref, we2, be2, wv1, bv1, wv2, bv2, wl1, bl1,
              wl2, bl2, out_ref):
    hb = h_ref[...]
    agg = jnp.dot(hb[0, 0] + hb[0, 1], we2[0:32, :])
    for p in range(1, 4):
        agg = agg + jnp.dot(hb[p, 0] + hb[p, 1], we2[32 * p:32 * p + 32, :])
    cnt = c_ref[0, :, 0:1] + c_ref[1, :, 0:1]
    agg = agg + cnt * be2[...]
    nin = n_ref[...]
    h2 = jnp.maximum(jnp.dot(agg, wv1[0:64, :]) + jnp.dot(nin, wv1[64:128, :])
                     + bv1[...], 0.0)
    nout = jnp.dot(h2, wv2[...]) + bv2[...]
    h3 = jnp.maximum(jnp.dot(nout, wl1[...]) + bl1[...], 0.0)
    out_ref[...] = jnp.dot(h3, wl2[...]) + bl2[...]


def _tc2(H, cnt, n_inp, we2, be2, wv1, bv1, wv2, bv2, wl1, bl1, wl2, bl2):
    bn = 2000
    grid = N_NODES // bn
    full = lambda a: pl.BlockSpec(a.shape, lambda ii: tuple(0 for _ in a.shape))
    return pl.pallas_call(
        _tc2_body,
        grid=(grid,),
        in_specs=[pl.BlockSpec((4, 2, bn, 32), lambda ii: (0, 0, ii, 0)),
                  pl.BlockSpec((2, bn, 16), lambda ii: (0, ii, 0)),
                  pl.BlockSpec((bn, 64), lambda ii: (ii, 0)),
                  full(we2), full(be2), full(wv1), full(bv1), full(wv2),
                  full(bv2), full(wl1), full(bl1), full(wl2), full(bl2)],
        out_specs=pl.BlockSpec((bn, 16), lambda ii: (ii, 0)),
        out_shape=jax.ShapeDtypeStruct((N_NODES, 16), jnp.float32),
    )(H, cnt, n_inp, we2, be2, wv1, bv1, wv2, bv2, wl1, bl1, wl2, bl2)


def kernel(theta, s, i, edge_index, Wi1, bi1, Wi2, bi2, We1, be1, We2, be2,
           Wv1, bv1, Wv2, bv2, Wu1, bu1, Wu2, bu2, Wl1, bl1, Wl2, bl2):
    B, P, A = theta.shape[0], theta.shape[1], theta.shape[2]
    n = B * P * A
    x = jnp.concatenate(
        [theta.reshape(n, -1), s.reshape(n, -1), i.reshape(n, -1),
         jnp.zeros((n, 15), jnp.float32)], axis=1)
    wi1p = jnp.concatenate([Wi1, jnp.zeros((15, Wi1.shape[1]), jnp.float32)],
                           axis=0)
    r2 = lambda b: b.reshape(1, -1)
    ei32 = edge_index.astype(jnp.int32)
    npad = N_EPAD - N_EDGES
    esrc2 = jnp.concatenate(
        [ei32[0], jnp.zeros((npad,), jnp.int32)]).reshape(-1, CHUNK)
    edst2 = jnp.concatenate(
        [ei32[1], jnp.full((npad,), N_NODES, jnp.int32)]).reshape(-1, CHUNK)
    cnt = _sc_count(edst2)
    n_inp, as0, as1, as2, as3, ad0, ad1, ad2, ad3 = _tc1(
        x, wi1p, r2(bi1), Wi2, r2(bi2), We1, r2(be1))
    H = _sc_edge((as0, as1, as2, as3), (ad0, ad1, ad2, ad3), esrc2, edst2)
    out = _tc2(H, cnt, n_inp, We2, r2(be2), Wv1, r2(bv1), Wv2, r2(bv2),
               Wl1, r2(bl1), Wl2, r2(bl2))
    return out.reshape(B, P, A, -1)
